# baseline (speedup 1.0000x reference)
import jax
import jax.numpy as jnp
from jax.experimental import pallas as pl

E_ENT = 100000
B = 128
ROW_BLK = 32


def _onehot_body(x_ref, out_ref):
    cols = jax.lax.broadcasted_iota(jnp.int32, (ROW_BLK, E_ENT), 1)
    out_ref[...] = (cols == x_ref[...]).astype(jnp.float32)


def kernel(input_x, input_r, e2triple, triple2e, r2triple, emb_table,
           W_ih, W_hh, b_ih, b_hh, W_lin, b_lin):
    x2d = input_x.astype(jnp.int32).reshape(B, 1)
    grid = (B // ROW_BLK,)
    return pl.pallas_call(
        _onehot_body,
        grid=grid,
        in_specs=[pl.BlockSpec((ROW_BLK, 1), lambda j: (j, 0))],
        out_specs=pl.BlockSpec((ROW_BLK, E_ENT), lambda j: (j, 0)),
        out_shape=jax.ShapeDtypeStruct((B, E_ENT), jnp.float32),
    )(x2d)
